# Initial kernel scaffold; baseline (speedup 1.0000x reference)
import jax
import jax.numpy as jnp
from jax.experimental import pallas as pl


def kernel(x, edge_index, W1, b1, W2, b2):
    raise NotImplementedError("write your pallas kernel here")


# speedup vs baseline: 4.5024x; 4.5024x over previous
import functools

import jax
import jax.numpy as jnp
from jax import lax
from jax.experimental import pallas as pl
from jax.experimental.pallas import tpu as pltpu
from jax.experimental.pallas import tpu_sc as plsc

N = 10000
E = 160000
D_IN = 128
D_H = 128
D_OUT = 64

NC = 2
NS = 16
NW = NC * NS

N_PAD = 10240
E_PAD = 163840
CHUNK = 128
CHUNKG = 64
CHUNKS = E_PAD // CHUNKG
CPT = CHUNKS // NW
ROWS_PT = N_PAD // NS
NBUF = 3

BM = 512


def _sc_mesh():
    return plsc.VectorSubcoreMesh(core_axis_name="c", subcore_axis_name="s")


def _make_agg():
    D = D_H

    @functools.partial(
        pl.kernel,
        out_type=jax.ShapeDtypeStruct((NC * N_PAD, D), jnp.float32),
        mesh=_sc_mesh(),
        scratch_types=(
            [pltpu.VMEM((CPT, CHUNKG), jnp.int32)] * 2
            + [pltpu.VMEM((CHUNKG, D), jnp.float32)] * NBUF
            + [pltpu.VMEM_SHARED((N_PAD, D), jnp.float32)]
            + [pltpu.SemaphoreType.DMA] * (2 * NBUF)
        ),
    )
    def agg_kernel(src2d, dst2d, y, zeros, out, idx_s, idx_d, *rest):
        rows = rest[:NBUF]
        acc = rest[NBUF]
        sem_g = rest[NBUF + 1:2 * NBUF + 1]
        sem_s = rest[2 * NBUF + 1:]
        c = lax.axis_index("c")
        s = lax.axis_index("s")
        wid = s * NC + c
        base = s * ROWS_PT
        pltpu.sync_copy(src2d.at[pl.ds(wid * CPT, CPT)], idx_s)
        pltpu.sync_copy(dst2d.at[pl.ds(wid * CPT, CPT)], idx_d)
        pltpu.sync_copy(zeros, rows[0])
        for k in range(ROWS_PT // CHUNKG):
            pltpu.sync_copy(rows[0], acc.at[pl.ds(base + k * CHUNKG, CHUNKG)])
        plsc.subcore_barrier()

        for b in range(2):
            pltpu.async_copy(y.at[idx_s.at[b]], rows[b], sem_g[b])

        def body(i, _):
            for b in range(NBUF):
                g = NBUF * i + b
                pltpu.make_async_copy(y.at[idx_s.at[0]], rows[b],
                                      sem_g[b]).wait()
                pltpu.async_copy(rows[b], acc.at[idx_d.at[g]], sem_s[b],
                                 add=True)
                nb = (b + 2) % NBUF

                @pl.when(g >= 1)
                def _drain():
                    pltpu.make_async_copy(
                        rows[nb], acc.at[idx_d.at[0]], sem_s[nb]).wait()

                pltpu.async_copy(y.at[idx_s.at[g + 2]], rows[nb], sem_g[nb])
            return 0

        lax.fori_loop(0, (CPT - 2) // NBUF, body, 0)
        for g, b in ((CPT - 2, 0), (CPT - 1, 1)):
            pltpu.make_async_copy(y.at[idx_s.at[0]], rows[b], sem_g[b]).wait()
            pltpu.async_copy(rows[b], acc.at[idx_d.at[g]], sem_s[b], add=True)
        for b in range(NBUF):
            pltpu.make_async_copy(rows[b], acc.at[idx_d.at[0]],
                                  sem_s[b]).wait()
        plsc.subcore_barrier()
        buf = rows[0]
        for k in range(ROWS_PT // CHUNKG):
            pltpu.sync_copy(acc.at[pl.ds(base + k * CHUNKG, CHUNKG)], buf)
            pltpu.sync_copy(
                buf, out.at[pl.ds(c * N_PAD + base + k * CHUNKG, CHUNKG)])

    return agg_kernel


_agg128 = _make_agg()


def _scale_of(degp_ref):
    d = degp_ref[0, :, 0:1] + degp_ref[1, :, 0:1] + 1.0
    return lax.rsqrt(d)


def _y1_body(x_ref, w_ref, degp_ref, o_ref):
    sc = _scale_of(degp_ref)
    o_ref[...] = sc * jnp.dot(x_ref[...], w_ref[...],
                              preferred_element_type=jnp.float32)


def _y2_body(p_ref, y1_ref, degp_ref, w_ref, b_ref, o_ref):
    sc = _scale_of(degp_ref)
    t = sc * (p_ref[0] + p_ref[1] + y1_ref[...]) + b_ref[...]
    h = jnp.maximum(t, 0.0)
    o_ref[...] = sc * jnp.dot(h, w_ref[...],
                              preferred_element_type=jnp.float32)


def _z_body(p_ref, y2_ref, degp_ref, b_ref, o_ref):
    sc = _scale_of(degp_ref)
    o_ref[...] = sc * (p_ref[0] + p_ref[1] + y2_ref[...]) + b_ref[...]


def _zzt_body(a_ref, b_ref, o_ref):
    o_ref[...] = lax.dot_general(
        a_ref[...], b_ref[...], (((1,), (1,)), ((), ())),
        preferred_element_type=jnp.float32)


_G = N_PAD // BM


def _row_spec(d):
    return pl.BlockSpec((BM, d), lambda i: (i, 0))


def _pair_spec(d):
    return pl.BlockSpec((2, BM, d), lambda i: (0, i, 0))


def _full_spec(shape):
    nd = len(shape)
    return pl.BlockSpec(shape, lambda i: (0,) * nd)


def _compute_y1(x_pad, W1, degp):
    return pl.pallas_call(
        _y1_body,
        grid=(_G,),
        in_specs=[_row_spec(D_IN), _full_spec((D_IN, D_H)), _pair_spec(16)],
        out_specs=_row_spec(D_H),
        out_shape=jax.ShapeDtypeStruct((N_PAD, D_H), jnp.float32),
    )(x_pad, W1, degp)


def _compute_y2(p1, y1, degp, W2p, b1):
    return pl.pallas_call(
        _y2_body,
        grid=(_G,),
        in_specs=[_pair_spec(D_H), _row_spec(D_H), _pair_spec(16),
                  _full_spec((D_H, D_H)), _full_spec((1, D_H))],
        out_specs=_row_spec(D_H),
        out_shape=jax.ShapeDtypeStruct((N_PAD, D_H), jnp.float32),
    )(p1, y1, degp, W2p, b1)


def _compute_z(p2, y2, degp, b2p):
    return pl.pallas_call(
        _z_body,
        grid=(_G,),
        in_specs=[_pair_spec(D_H), _row_spec(D_H), _pair_spec(16),
                  _full_spec((1, D_H))],
        out_specs=_row_spec(D_H),
        out_shape=jax.ShapeDtypeStruct((N_PAD, D_H), jnp.float32),
    )(p2, y2, degp, b2p)


def _zzt(z_pad):
    g_out = pl.cdiv(N, BM)
    return pl.pallas_call(
        _zzt_body,
        grid=(g_out, g_out),
        in_specs=[
            pl.BlockSpec((BM, D_H), lambda i, j: (i, 0)),
            pl.BlockSpec((BM, D_H), lambda i, j: (j, 0)),
        ],
        out_specs=pl.BlockSpec((BM, BM), lambda i, j: (i, j)),
        out_shape=jax.ShapeDtypeStruct((N, N), jnp.float32),
    )(z_pad, z_pad)


def kernel(x, edge_index, W1, b1, W2, b2):
    pad = jnp.full((E_PAD - E,), N, jnp.int32)
    src2d = jnp.concatenate([edge_index[0], pad]).reshape(CHUNKS, CHUNKG)
    dst2d = jnp.concatenate([edge_index[1], pad]).reshape(CHUNKS, CHUNKG)
    x_pad = jnp.pad(x, ((0, N_PAD - N), (0, 0)))

    zeros128 = jnp.zeros((CHUNKG, D_H), jnp.float32)
    ones_mat = jnp.ones((N_PAD, D_H), jnp.float32)
    W2p = jnp.pad(W2, ((0, 0), (0, D_H - D_OUT)))
    b2p = jnp.pad(b2, ((0, D_H - D_OUT),)).reshape(1, D_H)

    degp = (_agg128(src2d, dst2d, ones_mat, zeros128)
            .reshape(NC, N_PAD, D_H)[:, :, :16])
    y1 = _compute_y1(x_pad, W1, degp)
    p1 = _agg128(src2d, dst2d, y1, zeros128).reshape(NC, N_PAD, D_H)
    y2 = _compute_y2(p1, y1, degp, W2p, b1.reshape(1, D_H))
    p2 = _agg128(src2d, dst2d, y2, zeros128).reshape(NC, N_PAD, D_H)
    z_pad = _compute_z(p2, y2, degp, b2p)
    return _zzt(z_pad)
